# baseline (speedup 1.0000x reference)
import functools

import jax
import jax.numpy as jnp
from jax import lax
from jax.experimental import pallas as pl
from jax.experimental.pallas import tpu as pltpu
from jax.experimental.pallas import tpu_sc as plsc

NC = 2
NS = 16
NW = NC * NS
G = 128
NBUF = 4


@functools.lru_cache(maxsize=None)
def _build(n_rows: int, d: int):
  per_w = n_rows // NW
  ng = per_w // G

  mesh = plsc.VectorSubcoreMesh(
      core_axis_name="c", subcore_axis_name="s",
      num_cores=NC, num_subcores=NS)

  @functools.partial(
      pl.kernel,
      out_type=jax.ShapeDtypeStruct((n_rows, d), jnp.float32),
      mesh=mesh,
      scratch_types=[
          pltpu.VMEM((ng, G), jnp.int32),
          pltpu.VMEM((NBUF, G, d), jnp.float32),
          pltpu.SemaphoreType.DMA,
      ],
      compiler_params=pltpu.CompilerParams(use_tc_tiling_on_sc=False),
  )
  def gather_kernel(ids_hbm, emb_hbm, out_hbm, idx_v, rows_v, gsem):
    wid = lax.axis_index("s") * NC + lax.axis_index("c")
    base = wid * per_w

    pltpu.sync_copy(ids_hbm.at[wid], idx_v)

    for b in range(NBUF):
      pltpu.async_copy(emb_hbm.at[idx_v.at[b]], rows_v.at[b], gsem)

    @pl.loop(0, ng - NBUF, step=NBUF)
    def _steady(g0):
      for b in range(NBUF):
        g = g0 + b
        pltpu.make_async_copy(
            emb_hbm.at[idx_v.at[b]], rows_v.at[b], gsem).wait()
        pltpu.sync_copy(rows_v.at[b], out_hbm.at[pl.ds(base + g * G, G)])
        pltpu.async_copy(emb_hbm.at[idx_v.at[g + NBUF]], rows_v.at[b], gsem)

    for b in range(NBUF):
      g = ng - NBUF + b
      pltpu.make_async_copy(
          emb_hbm.at[idx_v.at[b]], rows_v.at[b], gsem).wait()
      pltpu.sync_copy(rows_v.at[b], out_hbm.at[pl.ds(base + g * G, G)])

  return gather_kernel


def kernel(input_ids, emb):
  bsz, seq = input_ids.shape
  _, d = emb.shape
  n_rows = bsz * seq
  ids3 = input_ids.astype(jnp.int32).reshape(NW, n_rows // (NW * G), G)
  out = _build(n_rows, d)(ids3, emb)
  return out.reshape(bsz, seq, d)
